# Initial kernel scaffold; baseline (speedup 1.0000x reference)
import jax
import jax.numpy as jnp
from jax.experimental import pallas as pl


def kernel(conv_w0, conv_b0, conv_w1, conv_b1, conv_w2, conv_b2, conv_w3, conv_b3, conv_w4, conv_b4, conv_w5, conv_b5, fc_w1, fc_b1, fc_w2, fc_b2, fc_w3, fc_b3, fc_w4, fc_b4, x):
    raise NotImplementedError("write your pallas kernel here")


# speedup vs baseline: 2.1246x; 2.1246x over previous
import functools

import jax
import jax.numpy as jnp
from jax.experimental import pallas as pl
from jax.experimental.pallas import tpu as pltpu

KT = 65
HALF = 16
N_OUT = 15


def _phases(feat):
    nd = feat.ndim
    pad = [(0, 0)] * (nd - 1) + [(HALF, HALF)]
    return (jnp.pad(feat[..., 0::2], pad), jnp.pad(feat[..., 1::2], pad))


def _l0_body(e_ref, o_ref, w_ref, b_ref, out_ref, col_ref, *, u, rb, cout):
    for r in range(rb):
        for k in range(KT + 1):
            src = o_ref if (k % 2) else e_ref
            col_ref[k, :] = src[r, pl.ds(k // 2, u)]
        y = jnp.dot(w_ref[...], col_ref[...], preferred_element_type=jnp.float32)
        y = jnp.maximum(y[:cout], y[cout:]) + b_ref[...]
        out_ref[r] = y.astype(jnp.bfloat16)


def _conv0(ep, op, w2, bias, *, rb=8):
    n_rows, wp = ep.shape
    u = wp - 2 * HALF
    cout = bias.shape[0]
    body = functools.partial(_l0_body, u=u, rb=rb, cout=cout)
    return pl.pallas_call(
        body,
        out_shape=jax.ShapeDtypeStruct((n_rows, cout, u), jnp.bfloat16),
        grid_spec=pltpu.PrefetchScalarGridSpec(
            num_scalar_prefetch=0,
            grid=(n_rows // rb,),
            in_specs=[
                pl.BlockSpec((rb, wp), lambda i: (i, 0)),
                pl.BlockSpec((rb, wp), lambda i: (i, 0)),
                pl.BlockSpec(w2.shape, lambda i: (0, 0)),
                pl.BlockSpec((cout, 1), lambda i: (0, 0)),
            ],
            out_specs=pl.BlockSpec((rb, cout, u), lambda i: (i, 0, 0)),
            scratch_shapes=[pltpu.VMEM((KT + 1, u), jnp.float32)],
        ),
        compiler_params=pltpu.CompilerParams(
            dimension_semantics=("parallel",),
            vmem_limit_bytes=48 * 1024 * 1024,
        ),
    )(ep, op, w2, bias)


def _conv_body(e_ref, o_ref, w_ref, b_ref, out_ref, col_ref, *, cin, u, rb, cout):
    for r in range(rb):
        for k in range(KT + 1):
            src = o_ref if (k % 2) else e_ref
            col_ref[k * cin:(k + 1) * cin, :] = src[r, :, pl.ds(k // 2, u)]
        y = jnp.dot(w_ref[...], col_ref[...], preferred_element_type=jnp.float32)
        y = jnp.maximum(y[:cout], y[cout:]) + b_ref[...]
        out_ref[r] = y.astype(jnp.bfloat16)


def _conv(ep, op, w2, bias, *, rb=8):
    n_rows, cin, wp = ep.shape
    u = wp - 2 * HALF
    cout = bias.shape[0]
    body = functools.partial(_conv_body, cin=cin, u=u, rb=rb, cout=cout)
    return pl.pallas_call(
        body,
        out_shape=jax.ShapeDtypeStruct((n_rows, cout, u), jnp.bfloat16),
        grid_spec=pltpu.PrefetchScalarGridSpec(
            num_scalar_prefetch=0,
            grid=(n_rows // rb,),
            in_specs=[
                pl.BlockSpec((rb, cin, wp), lambda i: (i, 0, 0)),
                pl.BlockSpec((rb, cin, wp), lambda i: (i, 0, 0)),
                pl.BlockSpec(w2.shape, lambda i: (0, 0)),
                pl.BlockSpec((cout, 1), lambda i: (0, 0)),
            ],
            out_specs=pl.BlockSpec((rb, cout, u), lambda i: (i, 0, 0)),
            scratch_shapes=[pltpu.VMEM(((KT + 1) * cin, u), jnp.bfloat16)],
        ),
        compiler_params=pltpu.CompilerParams(
            dimension_semantics=("parallel",),
            vmem_limit_bytes=48 * 1024 * 1024,
        ),
    )(ep, op, w2, bias)


def _fc_body(x_ref, w1_ref, b1_ref, w2_ref, b2_ref, w3_ref, b3_ref,
             w4_ref, b4_ref, o_ref):
    h = jnp.dot(x_ref[...], w1_ref[...], preferred_element_type=jnp.float32)
    h = h + b1_ref[...]
    h = jnp.dot(h, w2_ref[...], preferred_element_type=jnp.float32) + b2_ref[...]
    h = jnp.dot(h, w3_ref[...], preferred_element_type=jnp.float32) + b3_ref[...]
    h = jnp.dot(h, w4_ref[...], preferred_element_type=jnp.float32) + b4_ref[...]
    o_ref[...] = h


def _fc(x, w1, b1, w2, b2, w3, b3, w4, b4, *, rb=128):
    n, k = x.shape
    c1 = w1.shape[1]
    return pl.pallas_call(
        _fc_body,
        out_shape=jax.ShapeDtypeStruct((n, w4.shape[1]), jnp.float32),
        grid_spec=pltpu.PrefetchScalarGridSpec(
            num_scalar_prefetch=0,
            grid=(n // rb,),
            in_specs=[
                pl.BlockSpec((rb, k), lambda i: (i, 0)),
                pl.BlockSpec((k, c1), lambda i: (0, 0)),
                pl.BlockSpec((1, c1), lambda i: (0, 0)),
                pl.BlockSpec(w2.shape, lambda i: (0, 0)),
                pl.BlockSpec((1, w2.shape[1]), lambda i: (0, 0)),
                pl.BlockSpec(w3.shape, lambda i: (0, 0)),
                pl.BlockSpec((1, w3.shape[1]), lambda i: (0, 0)),
                pl.BlockSpec(w4.shape, lambda i: (0, 0)),
                pl.BlockSpec((1, w4.shape[1]), lambda i: (0, 0)),
            ],
            out_specs=pl.BlockSpec((rb, w4.shape[1]), lambda i: (i, 0)),
        ),
        compiler_params=pltpu.CompilerParams(
            dimension_semantics=("parallel",),
            vmem_limit_bytes=48 * 1024 * 1024,
        ),
    )(x, w1, b1, w2, b2, w3, b3, w4, b4)


def _stack_w(wmat, cin, dtype):
    cout = wmat.shape[0]
    w2 = jnp.zeros((2 * cout, (KT + 1) * cin), jnp.float32)
    w2 = w2.at[:cout, :KT * cin].set(wmat)
    w2 = w2.at[cout:, cin:(KT + 1) * cin].set(wmat)
    return w2.astype(dtype)


def kernel(conv_w0, conv_b0, conv_w1, conv_b1, conv_w2, conv_b2,
           conv_w3, conv_b3, conv_w4, conv_b4, conv_w5, conv_b5,
           fc_w1, fc_b1, fc_w2, fc_b2, fc_w3, fc_b3, fc_w4, fc_b4, x):
    n, _, h, w_img = x.shape
    rows = x.reshape(n * h, w_img)

    w0 = conv_w0.reshape(conv_w0.shape[0], KT, -1)[:, :, 0]
    ep, op = _phases(rows)
    feat = _conv0(ep, op, _stack_w(w0, 1, jnp.float32), conv_b0)

    for wmat, bias in ((conv_w1, conv_b1), (conv_w2, conv_b2),
                       (conv_w3, conv_b3), (conv_w4, conv_b4),
                       (conv_w5, conv_b5)):
        cin = feat.shape[1]
        ep, op = _phases(feat)
        feat = _conv(ep, op, _stack_w(wmat, cin, jnp.bfloat16), bias)

    r, cf, wf = feat.shape
    flat = feat.reshape(n, h * cf * wf)
    out = _fc(flat, fc_w1.astype(jnp.bfloat16), fc_b1,
              fc_w2, fc_b2, fc_w3, fc_b3, fc_w4, fc_b4)
    return out[:, :N_OUT]


# speedup vs baseline: 4.2145x; 1.9836x over previous
import functools

import jax
import jax.numpy as jnp
from jax.experimental import pallas as pl
from jax.experimental.pallas import tpu as pltpu

KT = 65
HALF = 16
N_OUT = 15


def _unpack(z):
    lo = jax.lax.bitcast_convert_type(z.astype(jnp.int16), jnp.bfloat16)
    hi = jax.lax.bitcast_convert_type((z >> 16).astype(jnp.int16), jnp.bfloat16)
    return lo, hi


def _pack_pairs(arr):
    shp = arr.shape[:-1] + (arr.shape[-1] // 2, 2)
    return jax.lax.bitcast_convert_type(arr.reshape(shp), jnp.int32)


def _conv_body(x_ref, w_ref, b_ref, out_ref, e_ref, o_ref, col_ref,
               *, cin, u, rb, cout, two_d):
    if two_d:
        z = jnp.zeros((rb, HALF), jnp.bfloat16)
    else:
        z = jnp.zeros((cin, HALF), jnp.bfloat16)
    e_ref[:, :HALF] = z
    e_ref[:, HALF + u:2 * HALF + u] = z
    o_ref[:, :HALF] = z
    o_ref[:, HALF + u:2 * HALF + u] = z
    if two_d:
        lo, hi = _unpack(x_ref[...])
        e_ref[:, HALF:HALF + u] = lo
        o_ref[:, HALF:HALF + u] = hi
    for r in range(rb):
        if not two_d:
            lo, hi = _unpack(x_ref[r])
            e_ref[:, HALF:HALF + u] = lo
            o_ref[:, HALF:HALF + u] = hi
        for k in range(KT + 1):
            src = o_ref if (k % 2) else e_ref
            if two_d:
                col_ref[k, :] = src[r, pl.ds(k // 2, u)]
            else:
                col_ref[k * cin:(k + 1) * cin, :] = src[:, pl.ds(k // 2, u)]
        y = jnp.dot(w_ref[...], col_ref[...], preferred_element_type=jnp.float32)
        y = jnp.maximum(y[:cout], y[cout:]) + b_ref[...]
        out_ref[r] = y.astype(jnp.bfloat16)


def _conv(xi, w2, bias, *, rb=8):
    two_d = xi.ndim == 2
    if two_d:
        n_rows, u = xi.shape
        cin = 1
        in_spec = pl.BlockSpec((rb, u), lambda i: (i, 0))
        e_shape = (rb, u + 2 * HALF)
    else:
        n_rows, cin, u = xi.shape
        in_spec = pl.BlockSpec((rb, cin, u), lambda i: (i, 0, 0))
        e_shape = (cin, u + 2 * HALF)
    cout = bias.shape[0]
    body = functools.partial(_conv_body, cin=cin, u=u, rb=rb, cout=cout,
                             two_d=two_d)
    return pl.pallas_call(
        body,
        out_shape=jax.ShapeDtypeStruct((n_rows, cout, u), jnp.bfloat16),
        grid_spec=pltpu.PrefetchScalarGridSpec(
            num_scalar_prefetch=0,
            grid=(n_rows // rb,),
            in_specs=[
                in_spec,
                pl.BlockSpec(w2.shape, lambda i: (0, 0)),
                pl.BlockSpec((cout, 1), lambda i: (0, 0)),
            ],
            out_specs=pl.BlockSpec((rb, cout, u), lambda i: (i, 0, 0)),
            scratch_shapes=[
                pltpu.VMEM(e_shape, jnp.bfloat16),
                pltpu.VMEM(e_shape, jnp.bfloat16),
                pltpu.VMEM(((KT + 1) * cin, u), jnp.bfloat16),
            ],
        ),
        compiler_params=pltpu.CompilerParams(
            dimension_semantics=("parallel",),
            vmem_limit_bytes=48 * 1024 * 1024,
        ),
    )(xi, w2, bias)


def _fc_body(x_ref, w1_ref, b1_ref, w2_ref, b2_ref, w3_ref, b3_ref,
             w4_ref, b4_ref, o_ref):
    h = jnp.dot(x_ref[...], w1_ref[...], preferred_element_type=jnp.float32)
    h = h + b1_ref[...]
    h = jnp.dot(h, w2_ref[...], preferred_element_type=jnp.float32) + b2_ref[...]
    h = jnp.dot(h, w3_ref[...], preferred_element_type=jnp.float32) + b3_ref[...]
    h = jnp.dot(h, w4_ref[...], preferred_element_type=jnp.float32) + b4_ref[...]
    o_ref[...] = h


def _fc(x, w1, b1, w2, b2, w3, b3, w4, b4, *, rb=128):
    n, k = x.shape
    c1 = w1.shape[1]
    return pl.pallas_call(
        _fc_body,
        out_shape=jax.ShapeDtypeStruct((n, w4.shape[1]), jnp.float32),
        grid_spec=pltpu.PrefetchScalarGridSpec(
            num_scalar_prefetch=0,
            grid=(n // rb,),
            in_specs=[
                pl.BlockSpec((rb, k), lambda i: (i, 0)),
                pl.BlockSpec((k, c1), lambda i: (0, 0)),
                pl.BlockSpec((1, c1), lambda i: (0, 0)),
                pl.BlockSpec(w2.shape, lambda i: (0, 0)),
                pl.BlockSpec((1, w2.shape[1]), lambda i: (0, 0)),
                pl.BlockSpec(w3.shape, lambda i: (0, 0)),
                pl.BlockSpec((1, w3.shape[1]), lambda i: (0, 0)),
                pl.BlockSpec(w4.shape, lambda i: (0, 0)),
                pl.BlockSpec((1, w4.shape[1]), lambda i: (0, 0)),
            ],
            out_specs=pl.BlockSpec((rb, w4.shape[1]), lambda i: (i, 0)),
        ),
        compiler_params=pltpu.CompilerParams(
            dimension_semantics=("parallel",),
            vmem_limit_bytes=48 * 1024 * 1024,
        ),
    )(x, w1, b1, w2, b2, w3, b3, w4, b4)


def _stack_w(wmat, cin, dtype):
    cout = wmat.shape[0]
    w2 = jnp.zeros((2 * cout, (KT + 1) * cin), jnp.float32)
    w2 = w2.at[:cout, :KT * cin].set(wmat)
    w2 = w2.at[cout:, cin:(KT + 1) * cin].set(wmat)
    return w2.astype(dtype)


def kernel(conv_w0, conv_b0, conv_w1, conv_b1, conv_w2, conv_b2,
           conv_w3, conv_b3, conv_w4, conv_b4, conv_w5, conv_b5,
           fc_w1, fc_b1, fc_w2, fc_b2, fc_w3, fc_b3, fc_w4, fc_b4, x):
    n, _, h, w_img = x.shape
    rows = x.reshape(n * h, w_img)

    w0 = conv_w0.reshape(conv_w0.shape[0], KT, -1)[:, :, 0]
    feat = _conv(_pack_pairs(rows.astype(jnp.bfloat16)),
                 _stack_w(w0, 1, jnp.bfloat16), conv_b0)

    for wmat, bias in ((conv_w1, conv_b1), (conv_w2, conv_b2),
                       (conv_w3, conv_b3), (conv_w4, conv_b4),
                       (conv_w5, conv_b5)):
        cin = feat.shape[1]
        feat = _conv(_pack_pairs(feat), _stack_w(wmat, cin, jnp.bfloat16), bias)

    r, cf, wf = feat.shape
    flat = feat.reshape(n, h * cf * wf)
    out = _fc(flat, fc_w1.astype(jnp.bfloat16), fc_b1,
              fc_w2, fc_b2, fc_w3, fc_b3, fc_w4, fc_b4)
    return out[:, :N_OUT]
